# Initial kernel scaffold; baseline (speedup 1.0000x reference)
import jax
import jax.numpy as jnp
from jax.experimental import pallas as pl


def kernel(x, offset_w, offset_b, mod_w, mod_b, weight, bias):
    raise NotImplementedError("write your pallas kernel here")


# speedup vs baseline: 8.6035x; 8.6035x over previous
import jax
import jax.numpy as jnp
from jax.experimental import pallas as pl
from jax.experimental.pallas import tpu as pltpu


def _deform_kernel(P, CHUNK, NC, PADR, H, W, K2):
    HW = H * W

    def body(xp, wof, bc, w2r, br, out_ref, pym, accr):
        blk = pl.program_id(1)
        p0 = blk * P
        ext = P + 16

        pvec = p0 + jax.lax.broadcasted_iota(jnp.int32, (P, 1), 0)
        wo_i = pvec % W
        ho_f = (pvec // W).astype(jnp.float32)
        wo_f = wo_i.astype(jnp.float32)

        slabs = [
            xp[0, pl.ds(pl.multiple_of(p0 + di * W, 8), ext), :]
            for di in range(3)
        ]
        conv = None
        for dj in range(3):
            a = None
            for di in range(3):
                t = jnp.dot(slabs[di], wof[di * 3 + dj],
                            preferred_element_type=jnp.float32)
                a = t if a is None else a + t
            asl = a[7 + dj:7 + dj + P, :]
            if dj == 0:
                asl = asl * (wo_i >= 1).astype(jnp.float32)
            elif dj == 2:
                asl = asl * (wo_i <= W - 2).astype(jnp.float32)
            conv = asl if conv is None else conv + asl
        conv = conv + bc[...]

        for k in range(K2):
            ki, kj = k // 3, k % 3
            dy = conv[:, 2 * k:2 * k + 1]
            dx = conv[:, 2 * k + 1:2 * k + 2]
            lg = conv[:, 2 * K2 + k:2 * K2 + k + 1]
            py = dy + (ho_f + (ki - 1))
            px = dx + (wo_f + (kj - 1))
            mk = 2.0 * jax.nn.sigmoid(lg)
            pym[k] = jnp.concatenate([py, px, mk], axis=1)

        accr[...] = jnp.zeros_like(accr)

        def step(i, carry):
            k = i // NC
            ci = i - k * NC
            v = pym[k]
            py = v[:, 0:1]
            px = v[:, 1:2]
            mk = v[:, 2:3]
            off = ci * CHUNK
            jl = jax.lax.broadcasted_iota(jnp.int32, (1, CHUNK), 1) + off
            r_row = (jl // W).astype(jnp.float32)
            c_row = (jl % W).astype(jnp.float32)
            ty = jnp.maximum(1.0 - jnp.abs(py - r_row), 0.0)
            tx = jnp.maximum(1.0 - jnp.abs(px - c_row), 0.0)
            s = (ty * tx).astype(jnp.bfloat16)
            xs = xp[0, pl.ds(pl.multiple_of(PADR + off, 8), CHUNK), :]
            g = jnp.dot(s, xs, preferred_element_type=jnp.float32)
            h = (g * mk).astype(jnp.bfloat16)
            accr[...] += jnp.dot(h, w2r[k], preferred_element_type=jnp.float32)
            return carry

        jax.lax.fori_loop(0, K2 * NC, step, 0)
        out_ref[0] = accr[...] + br[...]

    return body


def kernel(x, offset_w, offset_b, mod_w, mod_b, weight, bias):
    B, C, H, W = x.shape
    O = weight.shape[0]
    HW = H * W
    K2 = 9
    P = 512 if HW % 512 == 0 else HW
    CHUNK = 1024 if HW % 1024 == 0 else HW
    NC = HW // CHUNK
    NB = HW // P
    PADR = W + 8
    TOT = HW + 2 * PADR

    xf = jnp.transpose(x.reshape(B, C, HW), (0, 2, 1))
    xpad = jnp.pad(xf, ((0, 0), (PADR, PADR), (0, 0))).astype(jnp.bfloat16)
    wcat = jnp.concatenate([offset_w, mod_w], axis=0)
    wofs = jnp.transpose(wcat, (2, 3, 1, 0)).reshape(K2, C, 3 * K2)
    wofs = wofs.astype(jnp.bfloat16)
    bcat = jnp.concatenate([offset_b, mod_b]).reshape(1, 3 * K2)
    bcat = bcat.astype(jnp.float32)
    w2 = jnp.transpose(weight.reshape(O, C, K2), (2, 1, 0))
    w2 = w2.astype(jnp.bfloat16)
    b2 = bias.reshape(1, O).astype(jnp.float32)

    out = pl.pallas_call(
        _deform_kernel(P, CHUNK, NC, PADR, H, W, K2),
        out_shape=jax.ShapeDtypeStruct((B, HW, O), jnp.float32),
        grid=(B, NB),
        in_specs=[
            pl.BlockSpec((1, TOT, C), lambda b, i: (b, 0, 0)),
            pl.BlockSpec((K2, C, 3 * K2), lambda b, i: (0, 0, 0)),
            pl.BlockSpec((1, 3 * K2), lambda b, i: (0, 0)),
            pl.BlockSpec((K2, C, O), lambda b, i: (0, 0, 0)),
            pl.BlockSpec((1, O), lambda b, i: (0, 0)),
        ],
        out_specs=pl.BlockSpec((1, P, O), lambda b, i: (b, i, 0)),
        scratch_shapes=[
            pltpu.VMEM((K2, P, 3), jnp.float32),
            pltpu.VMEM((P, O), jnp.float32),
        ],
        compiler_params=pltpu.CompilerParams(
            dimension_semantics=("parallel", "arbitrary"),
        ),
        name="deform_conv2d_fused",
    )(xpad, wofs, bcat, w2, b2)

    return out.transpose(0, 2, 1).reshape(B, O, H, W)


# speedup vs baseline: 11.1501x; 1.2960x over previous
import jax
import jax.numpy as jnp
from jax.experimental import pallas as pl
from jax.experimental.pallas import tpu as pltpu


def _deform_kernel(P, CHUNK, NC, PADR, H, W, K2):
    HW = H * W
    RPC = CHUNK // W

    def body(xp, xtr, wof, bc, w2r, br, out_ref, cvt, accr):
        blk = pl.program_id(1)
        p0 = blk * P
        ext = P + 16

        pvec = p0 + jax.lax.broadcasted_iota(jnp.int32, (P, 1), 0)
        wo_i = pvec % W

        slabs = [
            xp[0, pl.ds(pl.multiple_of(p0 + di * W, 8), ext), :]
            for di in range(3)
        ]
        conv = None
        for dj in range(3):
            a = None
            for di in range(3):
                t = jnp.dot(slabs[di], wof[di * 3 + dj],
                            preferred_element_type=jnp.float32)
                a = t if a is None else a + t
            asl = a[7 + dj:7 + dj + P, :]
            if dj == 0:
                asl = asl * (wo_i >= 1).astype(jnp.float32)
            elif dj == 2:
                asl = asl * (wo_i <= W - 2).astype(jnp.float32)
            conv = asl if conv is None else conv + asl
        conv = conv + bc[...]
        cvt[0:3 * K2, :] = jnp.transpose(conv)

        prow = p0 + jax.lax.broadcasted_iota(jnp.int32, (1, P), 1)
        ho_row = (prow // W).astype(jnp.float32)
        wo_row = (prow % W).astype(jnp.float32)
        r_s = jax.lax.broadcasted_iota(jnp.int32, (RPC, 1), 0).astype(jnp.float32)
        c_s = jax.lax.broadcasted_iota(jnp.int32, (W, 1), 0).astype(jnp.float32)

        accr[...] = br[...] + jnp.zeros_like(accr)

        def step(i, carry):
            k = i // NC
            ci = i - k * NC
            ki = k // 3
            kj = k - 3 * ki
            b0 = pl.multiple_of(((2 * k) >> 3) << 3, 8)
            ch0 = cvt[pl.ds(b0, 8), :]
            ch0 = pltpu.roll(ch0, -((2 * k) & 7), axis=0)
            dy = ch0[0:1, :]
            dx = ch0[1:2, :]
            b1 = pl.multiple_of(((2 * K2 + k) >> 3) << 3, 8)
            ch1 = cvt[pl.ds(b1, 8), :]
            ch1 = pltpu.roll(ch1, -((2 * K2 + k) & 7), axis=0)
            mk = 2.0 * jax.nn.sigmoid(ch1[0:1, :])

            roff = (ci * RPC).astype(jnp.float32)
            py = dy + ho_row + (ki.astype(jnp.float32) - 1.0) - roff
            px = dx + wo_row + (kj.astype(jnp.float32) - 1.0)
            ty_s = jnp.maximum(1.0 - jnp.abs(py - r_s), 0.0)
            tx_s = jnp.maximum(1.0 - jnp.abs(px - c_s), 0.0)
            ty = jnp.broadcast_to(ty_s[:, None, :], (RPC, W, P))
            tx = jnp.broadcast_to(tx_s[None, :, :], (RPC, W, P))
            s = (ty * tx).astype(jnp.bfloat16).reshape(CHUNK, P)
            g = jnp.dot(xtr[0, ci], s, preferred_element_type=jnp.float32)
            h = (g * mk).astype(jnp.bfloat16)
            accr[...] += jnp.dot(w2r[k], h, preferred_element_type=jnp.float32)
            return carry

        jax.lax.fori_loop(0, K2 * NC, step, 0)
        out_ref[0] = accr[...]

    return body


def kernel(x, offset_w, offset_b, mod_w, mod_b, weight, bias):
    B, C, H, W = x.shape
    O = weight.shape[0]
    HW = H * W
    K2 = 9
    P = 512 if HW % 512 == 0 else HW
    CHUNK = 1024 if HW % 1024 == 0 else HW
    NC = HW // CHUNK
    NB = HW // P
    PADR = W + 8
    TOT = HW + 2 * PADR

    xf = jnp.transpose(x.reshape(B, C, HW), (0, 2, 1))
    xpad = jnp.pad(xf, ((0, 0), (PADR, PADR), (0, 0))).astype(jnp.bfloat16)
    xtr = jnp.transpose(x.reshape(B, C, NC, CHUNK), (0, 2, 1, 3))
    xtr = xtr.astype(jnp.bfloat16)
    wcat = jnp.concatenate([offset_w, mod_w], axis=0)
    wofs = jnp.transpose(wcat, (2, 3, 1, 0)).reshape(K2, C, 3 * K2)
    wofs = wofs.astype(jnp.bfloat16)
    bcat = jnp.concatenate([offset_b, mod_b]).reshape(1, 3 * K2)
    bcat = bcat.astype(jnp.float32)
    w2 = jnp.transpose(weight.reshape(O, C, K2), (2, 0, 1))
    w2 = w2.astype(jnp.bfloat16)
    b2 = bias.reshape(O, 1).astype(jnp.float32)

    out = pl.pallas_call(
        _deform_kernel(P, CHUNK, NC, PADR, H, W, K2),
        out_shape=jax.ShapeDtypeStruct((B, O, HW), jnp.float32),
        grid=(B, NB),
        in_specs=[
            pl.BlockSpec((1, TOT, C), lambda b, i: (b, 0, 0)),
            pl.BlockSpec((1, NC, C, CHUNK), lambda b, i: (b, 0, 0, 0)),
            pl.BlockSpec((K2, C, 3 * K2), lambda b, i: (0, 0, 0)),
            pl.BlockSpec((1, 3 * K2), lambda b, i: (0, 0)),
            pl.BlockSpec((K2, O, C), lambda b, i: (0, 0, 0)),
            pl.BlockSpec((O, 1), lambda b, i: (0, 0)),
        ],
        out_specs=pl.BlockSpec((1, O, P), lambda b, i: (b, 0, i)),
        scratch_shapes=[
            pltpu.VMEM((32, P), jnp.float32),
            pltpu.VMEM((O, P), jnp.float32),
        ],
        compiler_params=pltpu.CompilerParams(
            dimension_semantics=("parallel", "arbitrary"),
        ),
        name="deform_conv2d_fused",
    )(xpad, xtr, wofs, bcat, w2, b2)

    return out.reshape(B, O, H, W)


# speedup vs baseline: 14.0108x; 1.2566x over previous
import jax
import jax.numpy as jnp
from jax.experimental import pallas as pl
from jax.experimental.pallas import tpu as pltpu


def _deform_kernel(P, CHUNK, NC, PADR, H, W, K2):
    HW = H * W
    RPC = CHUNK // W

    def body(xp, xtr, wof, bc, w2r, br, out_ref, pymt, accr):
        blk = pl.program_id(1)
        p0 = blk * P
        ext = P + 16

        pvec = p0 + jax.lax.broadcasted_iota(jnp.int32, (P, 1), 0)
        wo_i = pvec % W

        slabs = [
            xp[0, pl.ds(pl.multiple_of(p0 + di * W, 8), ext), :]
            for di in range(3)
        ]
        conv = None
        for dj in range(3):
            a = None
            for di in range(3):
                t = jnp.dot(slabs[di], wof[di * 3 + dj],
                            preferred_element_type=jnp.float32)
                a = t if a is None else a + t
            asl = a[7 + dj:7 + dj + P, :]
            if dj == 0:
                asl = asl * (wo_i >= 1).astype(jnp.float32)
            elif dj == 2:
                asl = asl * (wo_i <= W - 2).astype(jnp.float32)
            conv = asl if conv is None else conv + asl
        conv = conv + bc[...]
        convt = jnp.transpose(conv)

        prow = p0 + jax.lax.broadcasted_iota(jnp.int32, (1, P), 1)
        ho_row = (prow // W).astype(jnp.float32)
        wo_row = (prow % W).astype(jnp.float32)
        r_s = jax.lax.broadcasted_iota(jnp.int32, (RPC, 1), 0).astype(jnp.float32)
        c_s = jax.lax.broadcasted_iota(jnp.int32, (W, 1), 0).astype(jnp.float32)

        for k in range(K2):
            ki, kj = k // 3, k % 3
            pymt[k, 0:1, :] = convt[2 * k:2 * k + 1, :] + ho_row + (ki - 1.0)
            pymt[k, 1:2, :] = convt[2 * k + 1:2 * k + 2, :] + wo_row + (kj - 1.0)
            pymt[k, 2:3, :] = 2.0 * jax.nn.sigmoid(
                convt[2 * K2 + k:2 * K2 + k + 1, :])

        accr[...] = br[...] + jnp.zeros_like(accr)

        def step(k, carry):
            v = pymt[k, 0:3, :]
            py = v[0:1, :]
            px = v[1:2, :]
            mk = v[2:3, :]
            tx_s = jnp.maximum(1.0 - jnp.abs(px - c_s), 0.0)
            txb = jnp.broadcast_to(tx_s[None, :, :], (RPC, W, P))
            for ci in range(NC):
                ty_s = jnp.maximum(
                    1.0 - jnp.abs(py - (r_s + ci * RPC)), 0.0)
                tyb = jnp.broadcast_to(ty_s[:, None, :], (RPC, W, P))
                s = (tyb * txb).astype(jnp.bfloat16).reshape(CHUNK, P)
                g = jnp.dot(xtr[0, ci], s, preferred_element_type=jnp.float32)
                h = (g * mk).astype(jnp.bfloat16)
                accr[...] += jnp.dot(w2r[k], h,
                                     preferred_element_type=jnp.float32)
            return carry

        jax.lax.fori_loop(0, K2, step, 0)
        out_ref[0] = accr[...]

    return body


def kernel(x, offset_w, offset_b, mod_w, mod_b, weight, bias):
    B, C, H, W = x.shape
    O = weight.shape[0]
    HW = H * W
    K2 = 9
    P = 512 if HW % 512 == 0 else HW
    CHUNK = 1024 if HW % 1024 == 0 else HW
    NC = HW // CHUNK
    NB = HW // P
    PADR = W + 8
    TOT = HW + 2 * PADR

    xf = jnp.transpose(x.reshape(B, C, HW), (0, 2, 1))
    xpad = jnp.pad(xf, ((0, 0), (PADR, PADR), (0, 0))).astype(jnp.bfloat16)
    xtr = jnp.transpose(x.reshape(B, C, NC, CHUNK), (0, 2, 1, 3))
    xtr = xtr.astype(jnp.bfloat16)
    wcat = jnp.concatenate([offset_w, mod_w], axis=0)
    wofs = jnp.transpose(wcat, (2, 3, 1, 0)).reshape(K2, C, 3 * K2)
    wofs = wofs.astype(jnp.bfloat16)
    bcat = jnp.concatenate([offset_b, mod_b]).reshape(1, 3 * K2)
    bcat = bcat.astype(jnp.float32)
    w2 = jnp.transpose(weight.reshape(O, C, K2), (2, 0, 1))
    w2 = w2.astype(jnp.bfloat16)
    b2 = bias.reshape(O, 1).astype(jnp.float32)

    out = pl.pallas_call(
        _deform_kernel(P, CHUNK, NC, PADR, H, W, K2),
        out_shape=jax.ShapeDtypeStruct((B, O, HW), jnp.float32),
        grid=(B, NB),
        in_specs=[
            pl.BlockSpec((1, TOT, C), lambda b, i: (b, 0, 0)),
            pl.BlockSpec((1, NC, C, CHUNK), lambda b, i: (b, 0, 0, 0)),
            pl.BlockSpec((K2, C, 3 * K2), lambda b, i: (0, 0, 0)),
            pl.BlockSpec((1, 3 * K2), lambda b, i: (0, 0)),
            pl.BlockSpec((K2, O, C), lambda b, i: (0, 0, 0)),
            pl.BlockSpec((O, 1), lambda b, i: (0, 0)),
        ],
        out_specs=pl.BlockSpec((1, O, P), lambda b, i: (b, 0, i)),
        scratch_shapes=[
            pltpu.VMEM((K2, 8, P), jnp.float32),
            pltpu.VMEM((O, P), jnp.float32),
        ],
        compiler_params=pltpu.CompilerParams(
            dimension_semantics=("parallel", "arbitrary"),
        ),
        name="deform_conv2d_fused",
    )(xpad, xtr, wofs, bcat, w2, b2)

    return out.reshape(B, O, H, W)


# speedup vs baseline: 14.0707x; 1.0043x over previous
import jax
import jax.numpy as jnp
from jax.experimental import pallas as pl
from jax.experimental.pallas import tpu as pltpu


def _deform_kernel(P, CHUNK, NC, PADR, H, W, K2):
    HW = H * W
    RPC = CHUNK // W

    def body(xp, xtr, wof, bc, w2r, br, out_ref, pymt, accr):
        blk = pl.program_id(1)
        p0 = blk * P
        ext = P + 16

        pvec = p0 + jax.lax.broadcasted_iota(jnp.int32, (P, 1), 0)
        wo_i = pvec % W

        slabs = [
            xp[0, pl.ds(pl.multiple_of(p0 + di * W, 8), ext), :]
            for di in range(3)
        ]
        conv = None
        for dj in range(3):
            a = None
            for di in range(3):
                t = jnp.dot(slabs[di], wof[di * 3 + dj],
                            preferred_element_type=jnp.float32)
                a = t if a is None else a + t
            asl = a[7 + dj:7 + dj + P, :]
            if dj == 0:
                asl = asl * (wo_i >= 1).astype(jnp.float32)
            elif dj == 2:
                asl = asl * (wo_i <= W - 2).astype(jnp.float32)
            conv = asl if conv is None else conv + asl
        conv = conv + bc[...]
        convt = jnp.transpose(conv)

        prow = p0 + jax.lax.broadcasted_iota(jnp.int32, (1, P), 1)
        ho_row = (prow // W).astype(jnp.float32)
        wo_row = (prow % W).astype(jnp.float32)
        r_s = jax.lax.broadcasted_iota(jnp.int32, (RPC, 1), 0).astype(jnp.float32)
        c_s = jax.lax.broadcasted_iota(jnp.int32, (W, 1), 0).astype(jnp.float32)

        for k in range(K2):
            ki, kj = k // 3, k % 3
            pymt[k, 0:1, :] = convt[2 * k:2 * k + 1, :] + ho_row + (ki - 1.0)
            pymt[k, 1:2, :] = convt[2 * k + 1:2 * k + 2, :] + wo_row + (kj - 1.0)
            pymt[k, 2:3, :] = 2.0 * jax.nn.sigmoid(
                convt[2 * K2 + k:2 * K2 + k + 1, :])

        accr[...] = br[...] + jnp.zeros_like(accr)
        xc = [xtr[0, ci] for ci in range(NC)]

        def step(k, carry):
            v = pymt[k, 0:3, :]
            py = v[0:1, :]
            px = v[1:2, :]
            mk = v[2:3, :]
            tx_s = jnp.maximum(1.0 - jnp.abs(px - c_s), 0.0)
            txb = jnp.broadcast_to(tx_s[None, :, :], (RPC, W, P))
            for ci in range(NC):
                ty_s = jnp.maximum(
                    1.0 - jnp.abs(py - (r_s + ci * RPC)), 0.0)
                tyb = jnp.broadcast_to(ty_s[:, None, :], (RPC, W, P))
                s = (tyb * txb).astype(jnp.bfloat16).reshape(CHUNK, P)
                g = jnp.dot(xc[ci], s, preferred_element_type=jnp.float32)
                h = (g * mk).astype(jnp.bfloat16)
                accr[...] += jnp.dot(w2r[k], h,
                                     preferred_element_type=jnp.float32)
            return carry

        jax.lax.fori_loop(0, K2, step, 0)
        out_ref[0] = accr[...]

    return body


def kernel(x, offset_w, offset_b, mod_w, mod_b, weight, bias):
    B, C, H, W = x.shape
    O = weight.shape[0]
    HW = H * W
    K2 = 9
    P = 512 if HW % 512 == 0 else HW
    CHUNK = 1024 if HW % 1024 == 0 else HW
    NC = HW // CHUNK
    NB = HW // P
    PADR = W + 8
    TOT = HW + 2 * PADR

    xf = jnp.transpose(x.reshape(B, C, HW), (0, 2, 1))
    xpad = jnp.pad(xf, ((0, 0), (PADR, PADR), (0, 0))).astype(jnp.bfloat16)
    xtr = jnp.transpose(x.reshape(B, C, NC, CHUNK), (0, 2, 1, 3))
    xtr = xtr.astype(jnp.bfloat16)
    wcat = jnp.concatenate([offset_w, mod_w], axis=0)
    wofs = jnp.transpose(wcat, (2, 3, 1, 0)).reshape(K2, C, 3 * K2)
    wofs = wofs.astype(jnp.bfloat16)
    bcat = jnp.concatenate([offset_b, mod_b]).reshape(1, 3 * K2)
    bcat = bcat.astype(jnp.float32)
    w2 = jnp.transpose(weight.reshape(O, C, K2), (2, 0, 1))
    w2 = w2.astype(jnp.bfloat16)
    b2 = bias.reshape(O, 1).astype(jnp.float32)

    out = pl.pallas_call(
        _deform_kernel(P, CHUNK, NC, PADR, H, W, K2),
        out_shape=jax.ShapeDtypeStruct((B, O, HW), jnp.float32),
        grid=(B, NB),
        in_specs=[
            pl.BlockSpec((1, TOT, C), lambda b, i: (b, 0, 0)),
            pl.BlockSpec((1, NC, C, CHUNK), lambda b, i: (b, 0, 0, 0)),
            pl.BlockSpec((K2, C, 3 * K2), lambda b, i: (0, 0, 0)),
            pl.BlockSpec((1, 3 * K2), lambda b, i: (0, 0)),
            pl.BlockSpec((K2, O, C), lambda b, i: (0, 0, 0)),
            pl.BlockSpec((O, 1), lambda b, i: (0, 0)),
        ],
        out_specs=pl.BlockSpec((1, O, P), lambda b, i: (b, 0, i)),
        scratch_shapes=[
            pltpu.VMEM((K2, 8, P), jnp.float32),
            pltpu.VMEM((O, P), jnp.float32),
        ],
        compiler_params=pltpu.CompilerParams(
            dimension_semantics=("parallel", "arbitrary"),
        ),
        name="deform_conv2d_fused",
    )(xpad, xtr, wofs, bcat, w2, b2)

    return out.reshape(B, O, H, W)


# speedup vs baseline: 19.1525x; 1.3612x over previous
import jax
import jax.numpy as jnp
from jax.experimental import pallas as pl
from jax.experimental.pallas import tpu as pltpu


def _deform_kernel(P, CHUNK, NC, PADR, H, W, K2):
    HW = H * W
    RPC = CHUNK // W

    def body(xp, xtr, wof, bc, w2r, br, out_ref, pymt, accr):
        blk = pl.program_id(1)
        p0 = blk * P
        ext = P + 16

        pvec = p0 + jax.lax.broadcasted_iota(jnp.int32, (P, 1), 0)
        wo_i = pvec % W

        slabs = [
            xp[0, pl.ds(pl.multiple_of(p0 + di * W, 8), ext), :]
            for di in range(3)
        ]
        conv = None
        for dj in range(3):
            a = None
            for di in range(3):
                t = jnp.dot(slabs[di], wof[di * 3 + dj],
                            preferred_element_type=jnp.float32)
                a = t if a is None else a + t
            asl = a[7 + dj:7 + dj + P, :]
            if dj == 0:
                asl = asl * (wo_i >= 1).astype(jnp.float32)
            elif dj == 2:
                asl = asl * (wo_i <= W - 2).astype(jnp.float32)
            conv = asl if conv is None else conv + asl
        conv = conv + bc[...]
        convt = jnp.transpose(conv)

        prow = p0 + jax.lax.broadcasted_iota(jnp.int32, (1, P), 1)
        ho_row = (prow // W).astype(jnp.float32)
        wo_row = (prow % W).astype(jnp.float32)
        r_s = jax.lax.broadcasted_iota(jnp.int32, (H, 1), 0).astype(jnp.float32)
        c_s = jax.lax.broadcasted_iota(jnp.int32, (W, 1), 0).astype(jnp.float32)

        for k in range(K2):
            ki, kj = k // 3, k % 3
            pymt[k, 0:1, :] = convt[2 * k:2 * k + 1, :] + ho_row + (ki - 1.0)
            pymt[k, 1:2, :] = convt[2 * k + 1:2 * k + 2, :] + wo_row + (kj - 1.0)
            pymt[k, 2:3, :] = 2.0 * jax.nn.sigmoid(
                convt[2 * K2 + k:2 * K2 + k + 1, :])

        accr[...] = br[...] + jnp.zeros_like(accr)
        xfull = xtr[0]

        def step(k, carry):
            v = pymt[k, 0:3, :]
            py = v[0:1, :]
            px = v[1:2, :]
            mk = v[2:3, :]
            ty_s = jnp.maximum(1.0 - jnp.abs(py - r_s), 0.0)
            tx_s = jnp.maximum(1.0 - jnp.abs(px - c_s), 0.0)
            tyb = jnp.broadcast_to(ty_s[:, None, :], (H, W, P))
            txb = jnp.broadcast_to(tx_s[None, :, :], (H, W, P))
            s = (tyb * txb).astype(jnp.bfloat16).reshape(HW, P)
            g = jnp.dot(xfull, s, preferred_element_type=jnp.float32)
            h = (g * mk).astype(jnp.bfloat16)
            accr[...] += jnp.dot(w2r[k], h,
                                 preferred_element_type=jnp.float32)
            return carry

        jax.lax.fori_loop(0, K2, step, 0)
        out_ref[0] = accr[...]

    return body


def kernel(x, offset_w, offset_b, mod_w, mod_b, weight, bias):
    B, C, H, W = x.shape
    O = weight.shape[0]
    HW = H * W
    K2 = 9
    P = 512 if HW % 512 == 0 else HW
    CHUNK = 1024 if HW % 1024 == 0 else HW
    NC = HW // CHUNK
    NB = HW // P
    PADR = W + 8
    TOT = HW + 2 * PADR

    xf = jnp.transpose(x.reshape(B, C, HW), (0, 2, 1))
    xpad = jnp.pad(xf, ((0, 0), (PADR, PADR), (0, 0))).astype(jnp.bfloat16)
    xtr = x.reshape(B, C, HW).astype(jnp.bfloat16)
    wcat = jnp.concatenate([offset_w, mod_w], axis=0)
    wofs = jnp.transpose(wcat, (2, 3, 1, 0)).reshape(K2, C, 3 * K2)
    wofs = wofs.astype(jnp.bfloat16)
    bcat = jnp.concatenate([offset_b, mod_b]).reshape(1, 3 * K2)
    bcat = bcat.astype(jnp.float32)
    w2 = jnp.transpose(weight.reshape(O, C, K2), (2, 0, 1))
    w2 = w2.astype(jnp.bfloat16)
    b2 = bias.reshape(O, 1).astype(jnp.float32)

    out = pl.pallas_call(
        _deform_kernel(P, CHUNK, NC, PADR, H, W, K2),
        out_shape=jax.ShapeDtypeStruct((B, O, HW), jnp.float32),
        grid=(B, NB),
        in_specs=[
            pl.BlockSpec((1, TOT, C), lambda b, i: (b, 0, 0)),
            pl.BlockSpec((1, C, HW), lambda b, i: (b, 0, 0)),
            pl.BlockSpec((K2, C, 3 * K2), lambda b, i: (0, 0, 0)),
            pl.BlockSpec((1, 3 * K2), lambda b, i: (0, 0)),
            pl.BlockSpec((K2, O, C), lambda b, i: (0, 0, 0)),
            pl.BlockSpec((O, 1), lambda b, i: (0, 0)),
        ],
        out_specs=pl.BlockSpec((1, O, P), lambda b, i: (b, 0, i)),
        scratch_shapes=[
            pltpu.VMEM((K2, 8, P), jnp.float32),
            pltpu.VMEM((O, P), jnp.float32),
        ],
        compiler_params=pltpu.CompilerParams(
            dimension_semantics=("parallel", "arbitrary"),
        ),
        name="deform_conv2d_fused",
    )(xpad, xtr, wofs, bcat, w2, b2)

    return out.reshape(B, O, H, W)


# speedup vs baseline: 19.2606x; 1.0056x over previous
import jax
import jax.numpy as jnp
from jax.experimental import pallas as pl
from jax.experimental.pallas import tpu as pltpu


def _deform_kernel(P, PADR, H, W, K2):
    HW = H * W

    def body(xp, xtr, wof, bc, w2r, br, out_ref, pymt, accr):
        blk = pl.program_id(1)
        p0 = blk * P
        ext = P + 16

        pvec = p0 + jax.lax.broadcasted_iota(jnp.int32, (P, 1), 0)
        wo_i = pvec % W

        slabs = [
            xp[0, pl.ds(pl.multiple_of(p0 + di * W, 8), ext), :]
            for di in range(3)
        ]
        conv = None
        for dj in range(3):
            a = None
            for di in range(3):
                t = jnp.dot(slabs[di], wof[di * 3 + dj],
                            preferred_element_type=jnp.float32)
                a = t if a is None else a + t
            asl = a[7 + dj:7 + dj + P, :]
            if dj == 0:
                asl = asl * (wo_i >= 1).astype(jnp.float32)
            elif dj == 2:
                asl = asl * (wo_i <= W - 2).astype(jnp.float32)
            conv = asl if conv is None else conv + asl
        conv = conv + bc[...]
        convt = jnp.transpose(conv)

        prow = p0 + jax.lax.broadcasted_iota(jnp.int32, (1, P), 1)
        ho_row = (prow // W).astype(jnp.float32)
        wo_row = (prow % W).astype(jnp.float32)
        r_s = jax.lax.broadcasted_iota(jnp.int32, (H, 1), 0).astype(jnp.float32)
        c_s = jax.lax.broadcasted_iota(jnp.int32, (W, 1), 0).astype(jnp.float32)

        for k in range(K2):
            ki, kj = k // 3, k % 3
            pymt[k, 0:1, :] = convt[2 * k:2 * k + 1, :] + ho_row + (ki - 1.0)
            pymt[k, 1:2, :] = convt[2 * k + 1:2 * k + 2, :] + wo_row + (kj - 1.0)
            pymt[k, 2:3, :] = 2.0 * jax.nn.sigmoid(
                convt[2 * K2 + k:2 * K2 + k + 1, :])

        accr[...] = br[...] + jnp.zeros_like(accr)
        xfull = xtr[0]

        def step(k, carry):
            v = pymt[k, 0:3, :]
            py = v[0:1, :]
            px = v[1:2, :]
            mk = v[2:3, :]
            ty_s = jnp.maximum(1.0 - jnp.abs(py - r_s), 0.0)
            tx_s = jnp.maximum(1.0 - jnp.abs(px - c_s), 0.0)
            tyb = jnp.broadcast_to(ty_s[:, None, :], (H, W, P))
            txb = jnp.broadcast_to(tx_s[None, :, :], (H, W, P))
            s = (tyb * txb).astype(jnp.bfloat16).reshape(HW, P)
            g = jnp.dot(xfull, s, preferred_element_type=jnp.float32)
            h = (g * mk).astype(jnp.bfloat16)
            accr[...] += jnp.dot(w2r[k], h,
                                 preferred_element_type=jnp.float32)
            return carry

        jax.lax.fori_loop(0, K2, step, 0)
        out_ref[0] = accr[...]

    return body


def kernel(x, offset_w, offset_b, mod_w, mod_b, weight, bias):
    B, C, H, W = x.shape
    O = weight.shape[0]
    HW = H * W
    K2 = 9
    P = 512 if HW % 512 == 0 else HW
    NB = HW // P
    PADR = W + 8
    TOT = HW + 2 * PADR

    xf = jnp.transpose(x.reshape(B, C, HW), (0, 2, 1))
    xpad = jnp.pad(xf, ((0, 0), (PADR, PADR), (0, 0))).astype(jnp.bfloat16)
    xtr = x.reshape(B, C, HW).astype(jnp.bfloat16)
    wcat = jnp.concatenate([offset_w, mod_w], axis=0)
    wofs = jnp.transpose(wcat, (2, 3, 1, 0)).reshape(K2, C, 3 * K2)
    wofs = wofs.astype(jnp.bfloat16)
    bcat = jnp.concatenate([offset_b, mod_b]).reshape(1, 3 * K2)
    bcat = bcat.astype(jnp.float32)
    w2 = jnp.transpose(weight.reshape(O, C, K2), (2, 0, 1))
    w2 = w2.astype(jnp.bfloat16)
    b2 = bias.reshape(O, 1).astype(jnp.float32)

    out = pl.pallas_call(
        _deform_kernel(P, PADR, H, W, K2),
        out_shape=jax.ShapeDtypeStruct((B, O, HW), jnp.float32),
        grid=(B, NB),
        in_specs=[
            pl.BlockSpec((1, TOT, C), lambda b, i: (b, 0, 0)),
            pl.BlockSpec((1, C, HW), lambda b, i: (b, 0, 0)),
            pl.BlockSpec((K2, C, 3 * K2), lambda b, i: (0, 0, 0)),
            pl.BlockSpec((1, 3 * K2), lambda b, i: (0, 0)),
            pl.BlockSpec((K2, O, C), lambda b, i: (0, 0, 0)),
            pl.BlockSpec((O, 1), lambda b, i: (0, 0)),
        ],
        out_specs=pl.BlockSpec((1, O, P), lambda b, i: (b, 0, i)),
        scratch_shapes=[
            pltpu.VMEM((K2, 8, P), jnp.float32),
            pltpu.VMEM((O, P), jnp.float32),
        ],
        compiler_params=pltpu.CompilerParams(
            dimension_semantics=("parallel", "arbitrary"),
        ),
        name="deform_conv2d_fused",
    )(xpad, xtr, wofs, bcat, w2, b2)

    return out.reshape(B, O, H, W)
